# baseline (speedup 1.0000x reference)
import jax
import jax.numpy as jnp
from jax.experimental import pallas as pl

D_MODEL = 128
MAX_LEN = 200
SEQ = 200

_BB = 128


def _bcast_kernel(w_ref, o_ref):
    o_ref[...] = jnp.broadcast_to(w_ref[...][None, :, :], o_ref.shape)


def kernel(x, pos_embedding_weight):
    bs, seq = x.shape
    grid = (bs // _BB,)
    out = pl.pallas_call(
        _bcast_kernel,
        grid=grid,
        in_specs=[pl.BlockSpec((seq, D_MODEL), lambda i: (0, 0))],
        out_specs=pl.BlockSpec((_BB, seq, D_MODEL), lambda i: (i, 0, 0)),
        out_shape=jax.ShapeDtypeStruct((bs, seq, D_MODEL), jnp.float32),
    )(pos_embedding_weight[:seq])
    return out


# speedup vs baseline: 1.0120x; 1.0120x over previous
import jax
import jax.numpy as jnp
from jax.experimental import pallas as pl

D_MODEL = 128
MAX_LEN = 200
SEQ = 200

_BB = 32


def _bcast_kernel(w_ref, o_ref):
    o_ref[...] = jnp.broadcast_to(w_ref[...][None, :, :], o_ref.shape)


def kernel(x, pos_embedding_weight):
    bs, seq = x.shape
    grid = (bs // _BB,)
    out = pl.pallas_call(
        _bcast_kernel,
        grid=grid,
        in_specs=[pl.BlockSpec((seq, D_MODEL), lambda i: (0, 0))],
        out_specs=pl.BlockSpec((_BB, seq, D_MODEL), lambda i: (i, 0, 0)),
        out_shape=jax.ShapeDtypeStruct((bs, seq, D_MODEL), jnp.float32),
    )(pos_embedding_weight[:seq])
    return out
